# baseline (speedup 1.0000x reference)
import functools
import math

import jax
import jax.numpy as jnp
from jax import lax
from jax.experimental import pallas as pl
from jax.experimental.pallas import tpu as pltpu
from jax.experimental.pallas import tpu_sc as plsc

_COST = 0.3
_SIM_LB = 0.6
_RATIO_LB = 0.1
_EPS = 1e-6

_NCH = 32


def _phase_a(x3, p_rows):
    R, _, D = x3.shape

    def body(x_ref, xp_ref, sq_ref, dot_ref):
        g = pl.program_id(0)
        xb = x_ref[...]
        sq_ref[...] = jnp.sum(xb * xb, axis=-1)
        d = jnp.sum(xb * xp_ref[...], axis=-1)
        dot_ref[...] = jnp.where(g == 0, 0.0, d)

    return pl.pallas_call(
        body,
        grid=(R // p_rows,),
        in_specs=[
            pl.BlockSpec((p_rows, 128, D), lambda g: (g, 0, 0)),
            pl.BlockSpec((p_rows, 128, D),
                         lambda g: (jnp.maximum(g - 1, 0), 0, 0)),
        ],
        out_specs=[
            pl.BlockSpec((p_rows, 128), lambda g: (g, 0)),
            pl.BlockSpec((p_rows, 128), lambda g: (g, 0)),
        ],
        out_shape=[
            jax.ShapeDtypeStruct((R, 128), jnp.float32),
            jax.ShapeDtypeStruct((R, 128), jnp.float32),
        ],
    )(x3, x3)


def _topk_mask(vals, k, idx, n):
    kb = lax.bitcast_convert_type(vals, jnp.int32)
    key = kb ^ (lax.shift_right_arithmetic(kb, 31) & jnp.int32(0x7FFFFFFF))

    def cnt_gt(t):
        return jnp.sum((key > t).astype(jnp.int32))

    def sbody(_, lohi):
        lo, hi = lohi
        mid = (lo >> 1) + (hi >> 1) + (lo & hi & 1)
        big = cnt_gt(mid) >= k
        return (jnp.where(big, mid, lo), jnp.where(big, hi, mid))

    lo, hi = lax.fori_loop(
        0, 33, sbody, (jnp.int32(-(2**31)), jnp.int32(2**31 - 1)))
    t = hi
    r = k - cnt_gt(t)
    eq = key == t

    def ibody(_, lohi):
        lo2, hi2 = lohi
        mid = (lo2 + hi2) >> 1
        good = jnp.sum((eq & (idx < mid)).astype(jnp.int32)) >= r
        return (jnp.where(good, lo2, mid), jnp.where(good, mid, hi2))

    _, bound = lax.fori_loop(0, 17, ibody, (jnp.int32(0), jnp.int32(n)))
    return (key > t) | (eq & (idx < bound))


def _phase_b(sq, sq_prev, sq_next, dot, dot_next, p_rows, k_merge, n_keep,
             trash):
    R = sq.shape[0]
    n = R * 128

    def body(sq_ref, sqp_ref, sqn_ref, dot_ref, dotn_ref,
             pos_ref, wn_ref):
        s = sq_ref[...]
        norm = jnp.sqrt(s)
        normp = jnp.sqrt(sqp_ref[...])
        idx = (lax.broadcasted_iota(jnp.int32, s.shape, 0) * 128
               + lax.broadcasted_iota(jnp.int32, s.shape, 1))
        first = idx < p_rows * 128
        sim = jnp.where(
            first, -1.0,
            dot_ref[...] / ((norm + _EPS) * (normp + _EPS)))
        msel = _topk_mask(sim, k_merge, idx, n)
        merge = msel & (sim > _SIM_LB)
        mf = merge.astype(jnp.float32)
        recv_f = jnp.concatenate(
            [mf[p_rows:], jnp.zeros((p_rows, 128), jnp.float32)], axis=0)
        recv = recv_f > 0.5
        imp_sq = jnp.where(
            recv,
            jnp.maximum(0.25 * (s + sqn_ref[...] + 2.0 * dotn_ref[...]), 0.0),
            s)
        imp = jnp.sqrt(imp_sq)
        score = jnp.where(merge, -jnp.inf, imp)
        keep = _topk_mask(score, n_keep, idx, n)
        wn_ref[...] = jnp.where(recv, 0.5, 0.0).astype(jnp.float32)

        kf = keep.astype(jnp.float32)
        ci = lax.broadcasted_iota(jnp.int32, (128, 128), 0)
        cj = lax.broadcasted_iota(jnp.int32, (128, 128), 1)
        ucol = (ci < cj).astype(jnp.float32)
        colpre = jax.lax.dot(kf, ucol, preferred_element_type=jnp.float32)
        ones = jnp.ones((128, 128), jnp.float32)
        rows = jax.lax.dot(kf, ones, preferred_element_type=jnp.float32)
        ri = lax.broadcasted_iota(jnp.int32, (R, R), 0)
        rj = lax.broadcasted_iota(jnp.int32, (R, R), 1)
        lrow = (rj < ri).astype(jnp.float32)
        rowpre = jax.lax.dot(lrow, rows, preferred_element_type=jnp.float32)
        posf = rowpre + colpre
        pos_ref[...] = jnp.where(
            keep, posf.astype(jnp.int32), jnp.int32(trash))

    return pl.pallas_call(
        body,
        out_shape=[
            jax.ShapeDtypeStruct((R, 128), jnp.int32),
            jax.ShapeDtypeStruct((R, 128), jnp.float32),
        ],
    )(sq, sq_prev, sq_next, dot, dot_next)


def _phase_m(x2, wn_col, p):
    n, d = x2.shape
    blk = p
    nb = n // blk

    def body(x_ref, xs_ref, w_ref, o_ref):
        g = pl.program_id(0)
        w = w_ref[...]
        val = (1.0 - w) * x_ref[...] + w * xs_ref[...]
        o_ref[...] = jnp.where(g == nb, 0.0, val)

    return pl.pallas_call(
        body,
        grid=(nb + 1,),
        in_specs=[
            pl.BlockSpec((blk, d), lambda g: (jnp.minimum(g, nb - 1), 0)),
            pl.BlockSpec((blk, d), lambda g: (jnp.minimum(g + 1, nb - 1), 0)),
            pl.BlockSpec((blk, 1), lambda g: (jnp.minimum(g, nb - 1), 0)),
        ],
        out_specs=pl.BlockSpec((blk, d), lambda g: (g, 0)),
        out_shape=jax.ShapeDtypeStruct((n + blk, d), jnp.float32),
    )(x2, x2, wn_col)


def _make_compact_sc(n, m, n_keep, trash):
    ch = n // _NCH
    npad = m - n_keep
    npad16 = ((npad + 15) // 16) * 16
    mesh = plsc.VectorSubcoreMesh(core_axis_name="c", subcore_axis_name="s")

    @functools.partial(
        pl.kernel,
        mesh=mesh,
        out_type=jax.ShapeDtypeStruct((m + 8,), jnp.int32),
        scratch_types=[
            pltpu.VMEM((ch,), jnp.int32),
            pltpu.VMEM((ch,), jnp.int32),
            pltpu.SemaphoreType.DMA,
        ],
    )
    def compact(pos_hbm, kidx_hbm, pos_v, idv, sem):
        wid = lax.axis_index("s") * 2 + lax.axis_index("c")
        t0 = wid * ch
        pltpu.sync_copy(pos_hbm.at[pl.ds(t0, ch)], pos_v)
        lane = lax.iota(jnp.int32, 16)

        def fill(j, _):
            idv[pl.ds(j * 16, 16)] = lane + (t0 + j * 16)
            return 0

        lax.fori_loop(0, ch // 16, fill, 0)
        pltpu.async_copy(idv, kidx_hbm.at[pos_v], sem).wait()

        if not npad:
            return

        @pl.when(wid == _NCH - 1)
        def _():
            def zb(j, _):
                e16 = lane + j * 16
                idv[pl.ds(j * 16, 16)] = jnp.full((16,), n, jnp.int32)
                pos_v[pl.ds(j * 16, 16)] = jnp.where(
                    e16 < npad, n_keep + e16, trash)
                return 0

            lax.fori_loop(0, npad16 // 16, zb, 0)
            pltpu.async_copy(idv.at[pl.ds(0, npad16)],
                             kidx_hbm.at[pos_v.at[pl.ds(0, npad16)]],
                             sem).wait()

    return compact


def _make_gather_sc(n, d, m, n_keep):
    slab = m // _NCH
    g = next(v for v in (72, 64, 48, 40, 24, 16, 8) if slab % v == 0)
    nb = slab // g
    mesh = plsc.VectorSubcoreMesh(core_axis_name="c", subcore_axis_name="s")

    @functools.partial(
        pl.kernel,
        mesh=mesh,
        out_type=jax.ShapeDtypeStruct((m, d), jnp.float32),
        scratch_types=[
            pltpu.VMEM((slab,), jnp.int32),
            pltpu.VMEM((g, d), jnp.float32),
            pltpu.SemaphoreType.DMA,
        ],
    )
    def gather(x_hbm, kidx_hbm, out_hbm, idxv, rowsb, sem):
        wid = lax.axis_index("s") * 2 + lax.axis_index("c")
        o0 = wid * slab
        pltpu.sync_copy(kidx_hbm.at[pl.ds(o0, slab)], idxv)

        def bbody(b, _):
            pltpu.async_copy(
                x_hbm.at[idxv.at[pl.ds(b * g, g)]], rowsb, sem).wait()
            pltpu.sync_copy(rowsb, out_hbm.at[pl.ds(o0 + b * g, g)])
            return 0

        lax.fori_loop(0, nb, bbody, 0)

    return gather


def kernel(x):
    f, p, d = x.shape
    n = f * p
    k_merge = int(_COST * n / 2)
    k_prune = int(_COST * n) - k_merge
    n_keep = n - k_merge - k_prune
    n_keep = max(n_keep, math.ceil(_RATIO_LB * n))
    pad = (p - n_keep % p) % p
    m = n_keep + pad
    rf = m // p
    p_rows = p // 128
    trash = m + 4

    x3 = x.reshape(n // 128, 128, d)
    sq, dot = _phase_a(x3, p_rows)
    zrows = jnp.zeros((p_rows, 128), jnp.float32)
    sq_prev = jnp.concatenate([zrows, sq[:-p_rows]], axis=0)
    sq_next = jnp.concatenate([sq[p_rows:], zrows], axis=0)
    dot_next = jnp.concatenate([dot[p_rows:], zrows], axis=0)
    pos, wn = _phase_b(
        sq, sq_prev, sq_next, dot, dot_next, p_rows, k_merge, n_keep, trash)

    merged = _phase_m(x.reshape(n, d), wn.reshape(n, 1), p)

    compact = _make_compact_sc(n, m, n_keep, trash)
    kidx = compact(pos.reshape(n))
    gather = _make_gather_sc(n, d, m, n_keep)
    out = gather(merged, kidx)
    return out.reshape(rf, p, d)


# speedup vs baseline: 1.0243x; 1.0243x over previous
import functools
import math

import jax
import jax.numpy as jnp
from jax import lax
from jax.experimental import pallas as pl
from jax.experimental.pallas import tpu as pltpu
from jax.experimental.pallas import tpu_sc as plsc

_COST = 0.3
_SIM_LB = 0.6
_RATIO_LB = 0.1
_EPS = 1e-6

_NCH = 32


def _phase_a(x3, p_rows):
    R, _, D = x3.shape

    def body(x_ref, sq_ref, dot_ref, prev_ref):
        g = pl.program_id(0)
        xb = x_ref[...]
        sq_ref[...] = jnp.sum(xb * xb, axis=-1)
        d = jnp.sum(xb * prev_ref[...], axis=-1)
        dot_ref[...] = jnp.where(g == 0, 0.0, d)
        prev_ref[...] = xb

    return pl.pallas_call(
        body,
        grid=(R // p_rows,),
        in_specs=[
            pl.BlockSpec((p_rows, 128, D), lambda g: (g, 0, 0)),
        ],
        out_specs=[
            pl.BlockSpec((p_rows, 128), lambda g: (g, 0)),
            pl.BlockSpec((p_rows, 128), lambda g: (g, 0)),
        ],
        out_shape=[
            jax.ShapeDtypeStruct((R, 128), jnp.float32),
            jax.ShapeDtypeStruct((R, 128), jnp.float32),
        ],
        scratch_shapes=[pltpu.VMEM((p_rows, 128, D), jnp.float32)],
    )(x3)


def _topk_mask(vals, k, idx, n):
    kb = lax.bitcast_convert_type(vals, jnp.int32)
    key = kb ^ (lax.shift_right_arithmetic(kb, 31) & jnp.int32(0x7FFFFFFF))

    def cnt_gt(t):
        return jnp.sum((key > t).astype(jnp.int32))

    def sbody(_, lohi):
        lo, hi = lohi
        mid = (lo >> 1) + (hi >> 1) + (lo & hi & 1)
        big = cnt_gt(mid) >= k
        return (jnp.where(big, mid, lo), jnp.where(big, hi, mid))

    lo, hi = lax.fori_loop(
        0, 33, sbody, (jnp.int32(-(2**31)), jnp.int32(2**31 - 1)))
    t = hi
    r = k - cnt_gt(t)
    eq = key == t

    def ibody(_, lohi):
        lo2, hi2 = lohi
        mid = (lo2 + hi2) >> 1
        good = jnp.sum((eq & (idx < mid)).astype(jnp.int32)) >= r
        return (jnp.where(good, lo2, mid), jnp.where(good, mid, hi2))

    _, bound = lax.fori_loop(0, 17, ibody, (jnp.int32(0), jnp.int32(n)))
    return (key > t) | (eq & (idx < bound))


def _phase_b(sq, sq_prev, sq_next, dot, dot_next, p_rows, k_merge, n_keep,
             trash):
    R = sq.shape[0]
    n = R * 128

    def body(sq_ref, sqp_ref, sqn_ref, dot_ref, dotn_ref,
             pos_ref, wn_ref):
        s = sq_ref[...]
        norm = jnp.sqrt(s)
        normp = jnp.sqrt(sqp_ref[...])
        idx = (lax.broadcasted_iota(jnp.int32, s.shape, 0) * 128
               + lax.broadcasted_iota(jnp.int32, s.shape, 1))
        first = idx < p_rows * 128
        sim = jnp.where(
            first, -1.0,
            dot_ref[...] / ((norm + _EPS) * (normp + _EPS)))
        msel = _topk_mask(sim, k_merge, idx, n)
        merge = msel & (sim > _SIM_LB)
        mf = merge.astype(jnp.float32)
        recv_f = jnp.concatenate(
            [mf[p_rows:], jnp.zeros((p_rows, 128), jnp.float32)], axis=0)
        recv = recv_f > 0.5
        imp_sq = jnp.where(
            recv,
            jnp.maximum(0.25 * (s + sqn_ref[...] + 2.0 * dotn_ref[...]), 0.0),
            s)
        imp = jnp.sqrt(imp_sq)
        score = jnp.where(merge, -jnp.inf, imp)
        keep = _topk_mask(score, n_keep, idx, n)
        wn_ref[...] = jnp.where(recv, 0.5, 0.0).astype(jnp.float32)

        kf = keep.astype(jnp.float32)
        ci = lax.broadcasted_iota(jnp.int32, (128, 128), 0)
        cj = lax.broadcasted_iota(jnp.int32, (128, 128), 1)
        ucol = (ci < cj).astype(jnp.float32)
        colpre = jax.lax.dot(kf, ucol, preferred_element_type=jnp.float32)
        ones = jnp.ones((128, 128), jnp.float32)
        rows = jax.lax.dot(kf, ones, preferred_element_type=jnp.float32)
        ri = lax.broadcasted_iota(jnp.int32, (R, R), 0)
        rj = lax.broadcasted_iota(jnp.int32, (R, R), 1)
        lrow = (rj < ri).astype(jnp.float32)
        rowpre = jax.lax.dot(lrow, rows, preferred_element_type=jnp.float32)
        posf = rowpre + colpre
        pos_ref[...] = jnp.where(
            keep, posf.astype(jnp.int32), jnp.int32(trash))

    return pl.pallas_call(
        body,
        out_shape=[
            jax.ShapeDtypeStruct((R, 128), jnp.int32),
            jax.ShapeDtypeStruct((R, 128), jnp.float32),
        ],
    )(sq, sq_prev, sq_next, dot, dot_next)


def _phase_m(x2, wn_col, p):
    n, d = x2.shape
    blk = p
    nb = n // blk

    def body(x_ref, w_ref, o_ref, prev_ref):
        g = pl.program_id(0)
        xb = x_ref[...]
        w = w_ref[...]
        val = (1.0 - w) * prev_ref[...] + w * xb
        o_ref[...] = jnp.where((g == 0) | (g == nb + 1), 0.0, val)
        prev_ref[...] = xb

    return pl.pallas_call(
        body,
        grid=(nb + 2,),
        in_specs=[
            pl.BlockSpec((blk, d), lambda g: (jnp.minimum(g, nb - 1), 0)),
            pl.BlockSpec((blk, 1),
                         lambda g: (jnp.clip(g - 1, 0, nb - 1), 0)),
        ],
        out_specs=pl.BlockSpec(
            (blk, d), lambda g: (jnp.clip(g - 1, 0, nb), 0)),
        out_shape=jax.ShapeDtypeStruct((n + blk, d), jnp.float32),
        scratch_shapes=[pltpu.VMEM((blk, d), jnp.float32)],
    )(x2, wn_col)


def _make_compact_sc(n, m, n_keep, trash):
    ch = n // _NCH
    npad = m - n_keep
    npad16 = ((npad + 15) // 16) * 16
    mesh = plsc.VectorSubcoreMesh(core_axis_name="c", subcore_axis_name="s")

    @functools.partial(
        pl.kernel,
        mesh=mesh,
        out_type=jax.ShapeDtypeStruct((m + 8,), jnp.int32),
        scratch_types=[
            pltpu.VMEM((ch,), jnp.int32),
            pltpu.VMEM((ch,), jnp.int32),
            pltpu.SemaphoreType.DMA,
        ],
    )
    def compact(pos_hbm, kidx_hbm, pos_v, idv, sem):
        wid = lax.axis_index("s") * 2 + lax.axis_index("c")
        t0 = wid * ch
        pltpu.sync_copy(pos_hbm.at[pl.ds(t0, ch)], pos_v)
        lane = lax.iota(jnp.int32, 16)

        def fill(j, _):
            idv[pl.ds(j * 16, 16)] = lane + (t0 + j * 16)
            return 0

        lax.fori_loop(0, ch // 16, fill, 0)
        pltpu.async_copy(idv, kidx_hbm.at[pos_v], sem).wait()

        if not npad:
            return

        @pl.when(wid == _NCH - 1)
        def _():
            def zb(j, _):
                e16 = lane + j * 16
                idv[pl.ds(j * 16, 16)] = jnp.full((16,), n, jnp.int32)
                pos_v[pl.ds(j * 16, 16)] = jnp.where(
                    e16 < npad, n_keep + e16, trash)
                return 0

            lax.fori_loop(0, npad16 // 16, zb, 0)
            pltpu.async_copy(idv.at[pl.ds(0, npad16)],
                             kidx_hbm.at[pos_v.at[pl.ds(0, npad16)]],
                             sem).wait()

    return compact


def _make_gather_sc(n, d, m, n_keep):
    slab = m // _NCH
    g = next(v for v in (72, 64, 48, 40, 24, 16, 8) if slab % v == 0)
    nb = slab // g
    mesh = plsc.VectorSubcoreMesh(core_axis_name="c", subcore_axis_name="s")

    @functools.partial(
        pl.kernel,
        mesh=mesh,
        out_type=jax.ShapeDtypeStruct((m, d), jnp.float32),
        scratch_types=[
            pltpu.VMEM((slab,), jnp.int32),
            pltpu.VMEM((g, d), jnp.float32),
            pltpu.VMEM((g, d), jnp.float32),
            pltpu.SemaphoreType.DMA,
            pltpu.SemaphoreType.DMA,
            pltpu.SemaphoreType.DMA,
            pltpu.SemaphoreType.DMA,
        ],
    )
    def gather(x_hbm, kidx_hbm, out_hbm, idxv,
               rows0, rows1, gs0, gs1, ws0, ws1):
        wid = lax.axis_index("s") * 2 + lax.axis_index("c")
        o0 = wid * slab
        pltpu.sync_copy(kidx_hbm.at[pl.ds(o0, slab)], idxv)
        bufs = (rows0, rows1)
        gsem = (gs0, gs1)
        wsem = (ws0, ws1)
        gcp = [None, None]
        wcp = [None, None]
        gcp[0] = pltpu.async_copy(
            x_hbm.at[idxv.at[pl.ds(0, g)]], bufs[0], gsem[0])
        for b in range(nb):
            cur = b & 1
            nxt = 1 - cur
            if b + 1 < nb:
                if wcp[nxt] is not None:
                    wcp[nxt].wait()
                gcp[nxt] = pltpu.async_copy(
                    x_hbm.at[idxv.at[pl.ds((b + 1) * g, g)]],
                    bufs[nxt], gsem[nxt])
            gcp[cur].wait()
            wcp[cur] = pltpu.async_copy(
                bufs[cur], out_hbm.at[pl.ds(o0 + b * g, g)], wsem[cur])
        wcp[(nb - 1) & 1].wait()
        if nb > 1:
            wcp[nb & 1].wait()

    return gather


def kernel(x):
    f, p, d = x.shape
    n = f * p
    k_merge = int(_COST * n / 2)
    k_prune = int(_COST * n) - k_merge
    n_keep = n - k_merge - k_prune
    n_keep = max(n_keep, math.ceil(_RATIO_LB * n))
    pad = (p - n_keep % p) % p
    m = n_keep + pad
    rf = m // p
    p_rows = p // 128
    trash = m + 4

    x3 = x.reshape(n // 128, 128, d)
    sq, dot = _phase_a(x3, p_rows)
    zrows = jnp.zeros((p_rows, 128), jnp.float32)
    sq_prev = jnp.concatenate([zrows, sq[:-p_rows]], axis=0)
    sq_next = jnp.concatenate([sq[p_rows:], zrows], axis=0)
    dot_next = jnp.concatenate([dot[p_rows:], zrows], axis=0)
    pos, wn = _phase_b(
        sq, sq_prev, sq_next, dot, dot_next, p_rows, k_merge, n_keep, trash)

    merged = _phase_m(x.reshape(n, d), wn.reshape(n, 1), p)

    compact = _make_compact_sc(n, m, n_keep, trash)
    kidx = compact(pos.reshape(n))
    gather = _make_gather_sc(n, d, m, n_keep)
    out = gather(merged, kidx)
    return out.reshape(rf, p, d)
